# baseline (speedup 1.0000x reference)
import functools

import jax
import jax.numpy as jnp
from jax import lax
from jax.experimental import pallas as pl
from jax.experimental.pallas import tpu as pltpu
from jax.experimental.pallas import tpu_sc as plsc

_NUM_FEATURES = 128
_ZMAX = 87

_NUM_CORES = 2
_NUM_SUBCORES = 16
_NW = _NUM_CORES * _NUM_SUBCORES

_CHUNK = 64
_NBUF = 6
_LOOK = 3


def _table_body(emb_ref, ec_ref, cw_ref, out_ref):
    out_ref[...] = emb_ref[...] + lax.dot_general(
        ec_ref[...], cw_ref[...],
        dimension_numbers=(((1,), (1,)), ((), ())),
        preferred_element_type=jnp.float32,
    )


def _build_table(element_embedding, config_weight, electron_config):
    return pl.pallas_call(
        _table_body,
        out_shape=jax.ShapeDtypeStruct((_ZMAX, _NUM_FEATURES), jnp.float32),
    )(element_embedding, electron_config, config_weight)


def _sc_gather(table, z2d):
    zrows, zcols = z2d.shape
    n = zrows * zcols
    b_per_w = n // _NW
    rows_per_w = b_per_w // zcols
    n_chunks = b_per_w // _CHUNK
    mesh = plsc.VectorSubcoreMesh(core_axis_name="c", subcore_axis_name="s")

    @functools.partial(
        pl.kernel,
        mesh=mesh,
        out_type=jax.ShapeDtypeStruct((n, _NUM_FEATURES), jnp.float32),
        scratch_types=[
            pltpu.VMEM_SHARED((_ZMAX, _NUM_FEATURES), jnp.float32),
            pltpu.VMEM_SHARED(
                (_NUM_SUBCORES, 3, _CHUNK, _NUM_FEATURES), jnp.float32),
            pltpu.VMEM((rows_per_w, zcols), jnp.int32),
        ]
        + [pltpu.VMEM((_CHUNK, _NUM_FEATURES), jnp.float32)] * _NBUF
        + [pltpu.SemaphoreType.DMA] * (2 * _NBUF + 6),
    )
    def k(table_hbm, idx_hbm, out_hbm, table_sp, spout, idx_v, *bs):
        bufs, gsems, osems = bs[:_NBUF], bs[_NBUF:2 * _NBUF], bs[2 * _NBUF:3 * _NBUF]
        bssem, bhsem = bs[3 * _NBUF:3 * _NBUF + 3], bs[3 * _NBUF + 3:]
        sid = lax.axis_index("s")
        wid = sid * _NUM_CORES + lax.axis_index("c")
        base = wid * b_per_w

        def start_gather(j, p):
            pltpu.async_copy(
                table_sp.at[idx_v.at[j]],
                bufs[p],
                gsems[p],
            )

        def wait_gather(p):
            pltpu.make_async_copy(
                out_hbm.at[pl.ds(0, _CHUNK)], bufs[p], gsems[p]
            ).wait()

        def wait_scatter(p):
            pltpu.make_async_copy(
                bufs[p], out_hbm.at[pl.ds(0, _CHUNK)], osems[p]
            ).wait()

        @pl.when(lax.axis_index("s") == 0)
        def _():
            pltpu.sync_copy(table_hbm, table_sp)

        pltpu.sync_copy(idx_hbm.at[pl.ds(wid * rows_per_w, rows_per_w)], idx_v)
        plsc.subcore_barrier()

        for j in range(_LOOK):
            start_gather(j, j % _NBUF)

        def wait_stage(r):
            pltpu.make_async_copy(
                bufs[0], spout.at[sid, r], bssem[r]
            ).wait()

        def wait_hbm_dma(r):
            pltpu.make_async_copy(
                spout.at[sid, r], out_hbm.at[pl.ds(0, _CHUNK)], bhsem[r]
            ).wait()

        def body(j, _):
            for p in range(_NBUF):
                @pl.when(j % _NBUF == p)
                def _(p=p):
                    wait_gather(p)
                    if p % 2 == 0:
                        pltpu.async_copy(
                            bufs[p],
                            out_hbm.at[pl.ds(base + j * _CHUNK, _CHUNK)],
                            osems[p],
                        )
                    else:
                        su = (p - 1) // 2
                        pr = (su - 1) % 3

                        @pl.when(j >= 7)
                        def _():
                            wait_hbm_dma(su)

                        pltpu.async_copy(bufs[p], spout.at[sid, su], bssem[su])

                        @pl.when(j >= 3)
                        def _():
                            wait_stage(pr)
                            pltpu.async_copy(
                                spout.at[sid, pr],
                                out_hbm.at[pl.ds(base + (j - 2) * _CHUNK,
                                                 _CHUNK)],
                                bhsem[pr],
                            )

            @pl.when(j + _LOOK < n_chunks)
            def _():
                for q in range(_NBUF):
                    @pl.when((j + _LOOK) % _NBUF == q)
                    def _(q=q):
                        if q % 2 == 0:
                            @pl.when(j + _LOOK >= _NBUF)
                            def _():
                                wait_scatter(q)

                        start_gather(j + _LOOK, q)

            return 0

        lax.fori_loop(0, n_chunks, body, 0)
        for p in range(0, _NBUF, 2):
            wait_scatter(p)
        last = n_chunks - 1
        lsu = ((last % _NBUF) - 1) // 2
        wait_stage(lsu)
        pltpu.async_copy(
            spout.at[sid, lsu],
            out_hbm.at[pl.ds(base + last * _CHUNK, _CHUNK)],
            bhsem[lsu],
        )
        for r in range(3):
            wait_hbm_dma(r)

    return k(table, z2d)


def kernel(Z, element_embedding, config_weight, electron_config):
    table = _build_table(element_embedding, config_weight, electron_config)
    out = _sc_gather(table, Z)
    return out.reshape(Z.shape + (_NUM_FEATURES,))


# speedup vs baseline: 1.3965x; 1.3965x over previous
import functools

import jax
import jax.numpy as jnp
from jax import lax
from jax.experimental import pallas as pl
from jax.experimental.pallas import tpu as pltpu
from jax.experimental.pallas import tpu_sc as plsc

_NUM_FEATURES = 128
_ZMAX = 87

_NUM_CORES = 2
_NUM_SUBCORES = 16
_NW = _NUM_CORES * _NUM_SUBCORES

_CHUNK = 128
_NBUF = 6
_LOOK = 3


def _table_body(emb_ref, ec_ref, cw_ref, out_ref):
    out_ref[...] = emb_ref[...] + lax.dot_general(
        ec_ref[...], cw_ref[...],
        dimension_numbers=(((1,), (1,)), ((), ())),
        preferred_element_type=jnp.float32,
    )


def _build_table(element_embedding, config_weight, electron_config):
    return pl.pallas_call(
        _table_body,
        out_shape=jax.ShapeDtypeStruct((_ZMAX, _NUM_FEATURES), jnp.float32),
    )(element_embedding, electron_config, config_weight)


def _sc_gather(table, z2d):
    zrows, zcols = z2d.shape
    n = zrows * zcols
    b_per_w = n // _NW
    rows_per_w = b_per_w // zcols
    n_chunks = b_per_w // _CHUNK
    mesh = plsc.VectorSubcoreMesh(core_axis_name="c", subcore_axis_name="s")

    @functools.partial(
        pl.kernel,
        mesh=mesh,
        out_type=jax.ShapeDtypeStruct((n, _NUM_FEATURES), jnp.float32),
        scratch_types=[
            pltpu.VMEM_SHARED((_ZMAX, _NUM_FEATURES), jnp.float32),
            pltpu.VMEM((rows_per_w, zcols), jnp.int32),
        ]
        + [pltpu.VMEM((_CHUNK, _NUM_FEATURES), jnp.float32)] * _NBUF
        + [pltpu.SemaphoreType.DMA] * (2 * _NBUF),
    )
    def k(table_hbm, idx_hbm, out_hbm, table_sp, idx_v, *bs):
        bufs, gsems, osems = bs[:_NBUF], bs[_NBUF:2 * _NBUF], bs[2 * _NBUF:]
        wid = lax.axis_index("s") * _NUM_CORES + lax.axis_index("c")
        base = wid * b_per_w

        def start_gather(j, p):
            pltpu.async_copy(
                table_sp.at[idx_v.at[j]],
                bufs[p],
                gsems[p],
            )

        def wait_gather(p):
            pltpu.make_async_copy(
                out_hbm.at[pl.ds(0, _CHUNK)], bufs[p], gsems[p]
            ).wait()

        def wait_scatter(p):
            pltpu.make_async_copy(
                bufs[p], out_hbm.at[pl.ds(0, _CHUNK)], osems[p]
            ).wait()

        @pl.when(lax.axis_index("s") == 0)
        def _():
            pltpu.sync_copy(table_hbm, table_sp)

        pltpu.sync_copy(idx_hbm.at[pl.ds(wid * rows_per_w, rows_per_w)], idx_v)
        plsc.subcore_barrier()

        for j in range(_LOOK):
            start_gather(j, j % _NBUF)

        def body(j, _):
            for p in range(_NBUF):
                @pl.when(j % _NBUF == p)
                def _(p=p):
                    wait_gather(p)
                    pltpu.async_copy(
                        bufs[p],
                        out_hbm.at[pl.ds(base + j * _CHUNK, _CHUNK)],
                        osems[p],
                    )

            @pl.when(j + _LOOK < n_chunks)
            def _():
                for q in range(_NBUF):
                    @pl.when((j + _LOOK) % _NBUF == q)
                    def _(q=q):
                        @pl.when(j + _LOOK >= _NBUF)
                        def _():
                            wait_scatter(q)

                        start_gather(j + _LOOK, q)

            return 0

        lax.fori_loop(0, n_chunks, body, 0)
        for p in range(_NBUF):
            wait_scatter(p)

    return k(table, z2d)


def kernel(Z, element_embedding, config_weight, electron_config):
    table = _build_table(element_embedding, config_weight, electron_config)
    out = _sc_gather(table, Z.reshape(-1, _CHUNK))
    return out.reshape(Z.shape + (_NUM_FEATURES,))


# speedup vs baseline: 1.3993x; 1.0020x over previous
import functools

import jax
import jax.numpy as jnp
from jax import lax
from jax.experimental import pallas as pl
from jax.experimental.pallas import tpu as pltpu
from jax.experimental.pallas import tpu_sc as plsc

_NUM_FEATURES = 128
_ZMAX = 87

_NUM_CORES = 2
_NUM_SUBCORES = 16
_NW = _NUM_CORES * _NUM_SUBCORES

_CHUNK = 64
_NBUF = 8
_LOOK = 4


def _table_body(emb_ref, ec_ref, cw_ref, out_ref):
    out_ref[...] = emb_ref[...] + lax.dot_general(
        ec_ref[...], cw_ref[...],
        dimension_numbers=(((1,), (1,)), ((), ())),
        preferred_element_type=jnp.float32,
    )


def _build_table(element_embedding, config_weight, electron_config):
    return pl.pallas_call(
        _table_body,
        out_shape=jax.ShapeDtypeStruct((_ZMAX, _NUM_FEATURES), jnp.float32),
    )(element_embedding, electron_config, config_weight)


def _sc_gather(table, z2d):
    zrows, zcols = z2d.shape
    n = zrows * zcols
    b_per_w = n // _NW
    rows_per_w = b_per_w // zcols
    n_chunks = b_per_w // _CHUNK
    mesh = plsc.VectorSubcoreMesh(core_axis_name="c", subcore_axis_name="s")

    @functools.partial(
        pl.kernel,
        mesh=mesh,
        out_type=jax.ShapeDtypeStruct((n, _NUM_FEATURES), jnp.float32),
        scratch_types=[
            pltpu.VMEM_SHARED((_ZMAX, _NUM_FEATURES), jnp.float32),
            pltpu.VMEM((rows_per_w, zcols), jnp.int32),
        ]
        + [pltpu.VMEM((_CHUNK, _NUM_FEATURES), jnp.float32)] * _NBUF
        + [pltpu.SemaphoreType.DMA] * (2 * _NBUF),
    )
    def k(table_hbm, idx_hbm, out_hbm, table_sp, idx_v, *bs):
        bufs, gsems, osems = bs[:_NBUF], bs[_NBUF:2 * _NBUF], bs[2 * _NBUF:]
        wid = lax.axis_index("s") * _NUM_CORES + lax.axis_index("c")
        base = wid * b_per_w

        def start_gather(j, p):
            pltpu.async_copy(
                table_sp.at[idx_v.at[j]],
                bufs[p],
                gsems[p],
            )

        def wait_gather(p):
            pltpu.make_async_copy(
                out_hbm.at[pl.ds(0, _CHUNK)], bufs[p], gsems[p]
            ).wait()

        def wait_scatter(p):
            pltpu.make_async_copy(
                bufs[p], out_hbm.at[pl.ds(0, _CHUNK)], osems[p]
            ).wait()

        @pl.when(lax.axis_index("s") == 0)
        def _():
            pltpu.sync_copy(table_hbm, table_sp)

        pltpu.sync_copy(idx_hbm.at[pl.ds(wid * rows_per_w, rows_per_w)], idx_v)
        plsc.subcore_barrier()

        for j in range(_LOOK):
            start_gather(j, j % _NBUF)

        def body(j, _):
            for p in range(_NBUF):
                @pl.when(j % _NBUF == p)
                def _(p=p):
                    wait_gather(p)
                    pltpu.async_copy(
                        bufs[p],
                        out_hbm.at[pl.ds(base + j * _CHUNK, _CHUNK)],
                        osems[p],
                    )

            @pl.when(j + _LOOK < n_chunks)
            def _():
                for q in range(_NBUF):
                    @pl.when((j + _LOOK) % _NBUF == q)
                    def _(q=q):
                        @pl.when(j + _LOOK >= _NBUF)
                        def _():
                            wait_scatter(q)

                        start_gather(j + _LOOK, q)

            return 0

        lax.fori_loop(0, n_chunks, body, 0)
        for p in range(_NBUF):
            wait_scatter(p)

    return k(table, z2d)


def kernel(Z, element_embedding, config_weight, electron_config):
    table = _build_table(element_embedding, config_weight, electron_config)
    out = _sc_gather(table, Z)
    return out.reshape(Z.shape + (_NUM_FEATURES,))


# speedup vs baseline: 1.4050x; 1.0041x over previous
import functools

import jax
import jax.numpy as jnp
from jax import lax
from jax.experimental import pallas as pl
from jax.experimental.pallas import tpu as pltpu
from jax.experimental.pallas import tpu_sc as plsc

_NUM_FEATURES = 128
_ZMAX = 87

_NUM_CORES = 2
_NUM_SUBCORES = 16
_NW = _NUM_CORES * _NUM_SUBCORES

_CHUNK = 64
_NBUF = 6
_LOOK = 3


def _table_body(emb_ref, ec_ref, cw_ref, out_ref):
    out_ref[...] = emb_ref[...] + lax.dot_general(
        ec_ref[...], cw_ref[...],
        dimension_numbers=(((1,), (1,)), ((), ())),
        preferred_element_type=jnp.float32,
    )


def _build_table(element_embedding, config_weight, electron_config):
    return pl.pallas_call(
        _table_body,
        out_shape=jax.ShapeDtypeStruct((_ZMAX, _NUM_FEATURES), jnp.float32),
    )(element_embedding, electron_config, config_weight)


def _sc_gather(table, z2d):
    zrows, zcols = z2d.shape
    n = zrows * zcols
    b_per_w = n // _NW
    rows_per_w = b_per_w // zcols
    n_chunks = b_per_w // _CHUNK
    mesh = plsc.VectorSubcoreMesh(core_axis_name="c", subcore_axis_name="s")

    @functools.partial(
        pl.kernel,
        mesh=mesh,
        out_type=jax.ShapeDtypeStruct((n, _NUM_FEATURES), jnp.float32),
        scratch_types=[
            pltpu.VMEM_SHARED((_ZMAX, _NUM_FEATURES), jnp.float32),
            pltpu.VMEM((rows_per_w, zcols), jnp.int32),
        ]
        + [pltpu.VMEM((_CHUNK, _NUM_FEATURES), jnp.float32)] * _NBUF
        + [pltpu.SemaphoreType.DMA] * (2 * _NBUF),
    )
    def k(table_hbm, idx_hbm, out_hbm, table_sp, idx_v, *bs):
        bufs, gsems, osems = bs[:_NBUF], bs[_NBUF:2 * _NBUF], bs[2 * _NBUF:]
        wid = lax.axis_index("s") * _NUM_CORES + lax.axis_index("c")
        base = wid * b_per_w

        def start_gather(j, p):
            pltpu.async_copy(
                table_sp.at[idx_v.at[j]],
                bufs[p],
                gsems[p],
            )

        def wait_gather(p):
            pltpu.make_async_copy(
                out_hbm.at[pl.ds(0, _CHUNK)], bufs[p], gsems[p]
            ).wait()

        def wait_scatter(p):
            pltpu.make_async_copy(
                bufs[p], out_hbm.at[pl.ds(0, _CHUNK)], osems[p]
            ).wait()

        @pl.when(lax.axis_index("s") == 0)
        def _():
            pltpu.sync_copy(table_hbm, table_sp)

        pltpu.sync_copy(idx_hbm.at[pl.ds(wid * rows_per_w, rows_per_w)], idx_v)
        plsc.subcore_barrier()

        for j in range(_LOOK):
            start_gather(j, j % _NBUF)

        def body(j, _):
            for p in range(_NBUF):
                @pl.when(j % _NBUF == p)
                def _(p=p):
                    wait_gather(p)
                    pltpu.async_copy(
                        bufs[p],
                        out_hbm.at[pl.ds(base + j * _CHUNK, _CHUNK)],
                        osems[p],
                    )

            @pl.when(j + _LOOK < n_chunks)
            def _():
                for q in range(_NBUF):
                    @pl.when((j + _LOOK) % _NBUF == q)
                    def _(q=q):
                        @pl.when(j + _LOOK >= _NBUF)
                        def _():
                            wait_scatter(q)

                        start_gather(j + _LOOK, q)

            return 0

        lax.fori_loop(0, n_chunks, body, 0)
        for p in range(_NBUF):
            wait_scatter(p)

    return k(table, z2d)


def kernel(Z, element_embedding, config_weight, electron_config):
    table = _build_table(element_embedding, config_weight, electron_config)
    out = _sc_gather(table, Z)
    return out.reshape(Z.shape + (_NUM_FEATURES,))


# speedup vs baseline: 1.4183x; 1.0095x over previous
import functools

import jax
import jax.numpy as jnp
from jax import lax
from jax.experimental import pallas as pl
from jax.experimental.pallas import tpu as pltpu
from jax.experimental.pallas import tpu_sc as plsc

_NUM_FEATURES = 128
_ZMAX = 87

_NUM_CORES = 2
_NUM_SUBCORES = 16
_NW = _NUM_CORES * _NUM_SUBCORES

_CHUNK = 64
_NBUF = 6
_LOOK = 3


def _table_body(emb_ref, ec_ref, cw_ref, out_ref):
    out_ref[...] = emb_ref[...] + lax.dot_general(
        ec_ref[...], cw_ref[...],
        dimension_numbers=(((1,), (1,)), ((), ())),
        preferred_element_type=jnp.float32,
    )


def _build_table(element_embedding, config_weight, electron_config):
    return pl.pallas_call(
        _table_body,
        out_shape=jax.ShapeDtypeStruct((_ZMAX, _NUM_FEATURES), jnp.float32),
    )(element_embedding, electron_config, config_weight)


def _sc_gather(table, z2d):
    zrows, zcols = z2d.shape
    n = zrows * zcols
    b_per_w = n // _NW
    rows_per_w = b_per_w // zcols
    n_chunks = b_per_w // _CHUNK
    mesh = plsc.VectorSubcoreMesh(core_axis_name="c", subcore_axis_name="s")

    @functools.partial(
        pl.kernel,
        mesh=mesh,
        out_type=jax.ShapeDtypeStruct((n, _NUM_FEATURES), jnp.float32),
        scratch_types=[
            pltpu.VMEM_SHARED((_ZMAX, _NUM_FEATURES), jnp.float32),
            pltpu.VMEM((rows_per_w, zcols), jnp.int32),
        ]
        + [pltpu.VMEM((_CHUNK, _NUM_FEATURES), jnp.float32)] * _NBUF
        + [pltpu.SemaphoreType.DMA] * (2 * _NBUF + 1),
    )
    def k(table_hbm, idx_hbm, out_hbm, table_sp, idx_v, *bs):
        bufs, gsems = bs[:_NBUF], bs[_NBUF:2 * _NBUF]
        osems, tsem = bs[2 * _NBUF:3 * _NBUF], bs[3 * _NBUF]
        wid = lax.axis_index("s") * _NUM_CORES + lax.axis_index("c")
        base = wid * b_per_w

        def start_gather(j, p):
            pltpu.async_copy(
                table_sp.at[idx_v.at[j]],
                bufs[p],
                gsems[p],
            )

        def wait_gather(p):
            pltpu.make_async_copy(
                out_hbm.at[pl.ds(0, _CHUNK)], bufs[p], gsems[p]
            ).wait()

        def wait_scatter(p):
            pltpu.make_async_copy(
                bufs[p], out_hbm.at[pl.ds(0, _CHUNK)], osems[p]
            ).wait()

        @pl.when(lax.axis_index("s") == 0)
        def _():
            pltpu.async_copy(table_hbm, table_sp, tsem)

        pltpu.sync_copy(idx_hbm.at[pl.ds(wid * rows_per_w, rows_per_w)], idx_v)

        @pl.when(lax.axis_index("s") == 0)
        def _():
            pltpu.make_async_copy(table_hbm, table_sp, tsem).wait()

        plsc.subcore_barrier()

        for j in range(_LOOK):
            start_gather(j, j % _NBUF)

        def body(j, _):
            for p in range(_NBUF):
                @pl.when(j % _NBUF == p)
                def _(p=p):
                    wait_gather(p)
                    pltpu.async_copy(
                        bufs[p],
                        out_hbm.at[pl.ds(base + j * _CHUNK, _CHUNK)],
                        osems[p],
                    )

            @pl.when(j + _LOOK < n_chunks)
            def _():
                for q in range(_NBUF):
                    @pl.when((j + _LOOK) % _NBUF == q)
                    def _(q=q):
                        @pl.when(j + _LOOK >= _NBUF)
                        def _():
                            wait_scatter(q)

                        start_gather(j + _LOOK, q)

            return 0

        lax.fori_loop(0, n_chunks, body, 0)
        for p in range(_NBUF):
            wait_scatter(p)

    return k(table, z2d)


def kernel(Z, element_embedding, config_weight, electron_config):
    table = _build_table(element_embedding, config_weight, electron_config)
    out = _sc_gather(table, Z)
    return out.reshape(Z.shape + (_NUM_FEATURES,))


# speedup vs baseline: 1.4201x; 1.0013x over previous
import functools

import jax
import jax.numpy as jnp
from jax import lax
from jax.experimental import pallas as pl
from jax.experimental.pallas import tpu as pltpu
from jax.experimental.pallas import tpu_sc as plsc

_NUM_FEATURES = 128
_ZMAX = 87

_NUM_CORES = 2
_NUM_SUBCORES = 16
_NW = _NUM_CORES * _NUM_SUBCORES

_CHUNK = 64
_NBUF = 6
_LOOK = 3


def _table_body(emb_ref, ec_ref, cw_ref, out_ref):
    out_ref[...] = emb_ref[...] + lax.dot_general(
        ec_ref[...], cw_ref[...],
        dimension_numbers=(((1,), (1,)), ((), ())),
        preferred_element_type=jnp.float32,
    )


def _build_table(element_embedding, config_weight, electron_config):
    return pl.pallas_call(
        _table_body,
        out_shape=jax.ShapeDtypeStruct((_ZMAX, _NUM_FEATURES), jnp.float32),
    )(element_embedding, electron_config, config_weight)


def _sc_gather(table, z2d):
    zrows, zcols = z2d.shape
    n = zrows * zcols
    b_per_w = n // _NW
    rows_per_w = b_per_w // zcols
    n_chunks = b_per_w // _CHUNK
    mesh = plsc.VectorSubcoreMesh(core_axis_name="c", subcore_axis_name="s")

    @functools.partial(
        pl.kernel,
        mesh=mesh,
        out_type=jax.ShapeDtypeStruct((n, _NUM_FEATURES), jnp.float32),
        scratch_types=[
            pltpu.VMEM_SHARED((_ZMAX, _NUM_FEATURES), jnp.float32),
            pltpu.VMEM((rows_per_w, zcols), jnp.int32),
        ]
        + [pltpu.VMEM((_CHUNK, _NUM_FEATURES), jnp.float32)] * _NBUF
        + [pltpu.SemaphoreType.DMA] * (2 * _NBUF + 1),
    )
    def k(table_hbm, idx_hbm, out_hbm, table_sp, idx_v, *bs):
        bufs, gsems = bs[:_NBUF], bs[_NBUF:2 * _NBUF]
        osems, tsem = bs[2 * _NBUF:3 * _NBUF], bs[3 * _NBUF]
        wid = lax.axis_index("s") * _NUM_CORES + lax.axis_index("c")
        base = wid * b_per_w

        def start_gather(j, p):
            pltpu.async_copy(
                table_sp.at[idx_v.at[j]],
                bufs[p],
                gsems[p],
            )

        def wait_gather(p):
            pltpu.make_async_copy(
                out_hbm.at[pl.ds(0, _CHUNK)], bufs[p], gsems[p]
            ).wait()

        def wait_scatter(p):
            pltpu.make_async_copy(
                bufs[p], out_hbm.at[pl.ds(0, _CHUNK)], osems[p]
            ).wait()

        @pl.when(lax.axis_index("s") == 0)
        def _():
            pltpu.async_copy(table_hbm, table_sp, tsem)

        head = 8
        pltpu.sync_copy(
            idx_hbm.at[pl.ds(wid * rows_per_w, head)], idx_v.at[pl.ds(0, head)]
        )

        @pl.when(lax.axis_index("s") == 0)
        def _():
            pltpu.make_async_copy(table_hbm, table_sp, tsem).wait()

        plsc.subcore_barrier()

        for j in range(_LOOK):
            start_gather(j, j % _NBUF)

        pltpu.sync_copy(
            idx_hbm.at[pl.ds(wid * rows_per_w + head, rows_per_w - head)],
            idx_v.at[pl.ds(head, rows_per_w - head)],
        )

        def body(j, _):
            for p in range(_NBUF):
                @pl.when(j % _NBUF == p)
                def _(p=p):
                    wait_gather(p)
                    pltpu.async_copy(
                        bufs[p],
                        out_hbm.at[pl.ds(base + j * _CHUNK, _CHUNK)],
                        osems[p],
                    )

            @pl.when(j + _LOOK < n_chunks)
            def _():
                for q in range(_NBUF):
                    @pl.when((j + _LOOK) % _NBUF == q)
                    def _(q=q):
                        @pl.when(j + _LOOK >= _NBUF)
                        def _():
                            wait_scatter(q)

                        start_gather(j + _LOOK, q)

            return 0

        lax.fori_loop(0, n_chunks, body, 0)
        for p in range(_NBUF):
            wait_scatter(p)

    return k(table, z2d)


def kernel(Z, element_embedding, config_weight, electron_config):
    table = _build_table(element_embedding, config_weight, electron_config)
    out = _sc_gather(table, Z)
    return out.reshape(Z.shape + (_NUM_FEATURES,))
